# baseline (speedup 1.0000x reference)
import jax
import jax.numpy as jnp
from jax.experimental import pallas as pl

_N_NODES = 50000
_N_EDGES = 1600000
_N_UNARY = 8
_N_BINARY = 2

_U_ROWS = (_N_NODES * _N_UNARY) // 128
_B_ROWS = (_N_EDGES * _N_BINARY) // 128
_B_BLOCK = 5000
_GRID = _B_ROWS // _B_BLOCK


def _copy_body(u_ref, b_ref, uo_ref, bo_ref):
    bo_ref[...] = b_ref[...]

    @pl.when(pl.program_id(0) == 0)
    def _():
        uo_ref[...] = u_ref[...]


def kernel(unary, binary, index1, index2):
    u2 = unary.reshape(_U_ROWS, 128)
    b2 = binary.reshape(_B_ROWS, 128)
    uo, bo = pl.pallas_call(
        _copy_body,
        grid=(_GRID,),
        in_specs=[
            pl.BlockSpec((_U_ROWS, 128), lambda i: (0, 0)),
            pl.BlockSpec((_B_BLOCK, 128), lambda i: (i, 0)),
        ],
        out_specs=[
            pl.BlockSpec((_U_ROWS, 128), lambda i: (0, 0)),
            pl.BlockSpec((_B_BLOCK, 128), lambda i: (i, 0)),
        ],
        out_shape=[
            jax.ShapeDtypeStruct((_U_ROWS, 128), unary.dtype),
            jax.ShapeDtypeStruct((_B_ROWS, 128), binary.dtype),
        ],
    )(u2, b2)
    return (uo.reshape(unary.shape), bo.reshape(binary.shape))
